# baseline (speedup 1.0000x reference)
import functools

import jax
import jax.numpy as jnp
from jax.experimental import pallas as pl
from jax.experimental.pallas import tpu as pltpu

_PARALLEL = pltpu.CompilerParams(dimension_semantics=("parallel",))

_B, _P, _IN, _OUT = 16, 2048, 3, 13
_NEG = -1e30
_MS = [410, 123, 37, 12, 4]
_RS = [0.05, 0.1, 0.2, 0.4, 0.8]
_MAXF = 0x7F7FFFFF


def _fps_body(px_ref, py_ref, pz_ref, qx_ref, qy_ref, qz_ref, *, m, n):
    px = px_ref[...]
    py = py_ref[...]
    pz = pz_ref[...]
    b = px.shape[0]
    lane = jax.lax.broadcasted_iota(jnp.int32, (b, n), 1)
    lane_m = jax.lax.broadcasted_iota(jnp.int32, (b, m), 1)
    lx = px[:, 0:1]
    ly = py[:, 0:1]
    lz = pz[:, 0:1]
    qx0 = jnp.where(lane_m == 0, lx, 0.0)
    qy0 = jnp.where(lane_m == 0, ly, 0.0)
    qz0 = jnp.where(lane_m == 0, lz, 0.0)
    d0 = jnp.full((b, n), jnp.inf, dtype=jnp.float32)

    def body(i, carry):
        dists, cx, cy, cz, qx, qy, qz = carry
        d = (px - cx) ** 2 + (py - cy) ** 2 + (pz - cz) ** 2
        dists = jnp.minimum(dists, d)
        mx = jnp.max(dists, axis=1, keepdims=True)
        cand = jnp.where(dists == mx, lane, n)
        nxt = jnp.min(cand, axis=1, keepdims=True)
        oh = lane == nxt
        nx = jnp.sum(jnp.where(oh, px, 0.0), axis=1, keepdims=True)
        ny = jnp.sum(jnp.where(oh, py, 0.0), axis=1, keepdims=True)
        nz = jnp.sum(jnp.where(oh, pz, 0.0), axis=1, keepdims=True)
        qx = jnp.where(lane_m == i, nx, qx)
        qy = jnp.where(lane_m == i, ny, qy)
        qz = jnp.where(lane_m == i, nz, qz)
        return dists, nx, ny, nz, qx, qy, qz

    _, _, _, _, qx, qy, qz = jax.lax.fori_loop(
        1, m, body, (d0, lx, ly, lz, qx0, qy0, qz0)
    )
    qx_ref[...] = qx
    qy_ref[...] = qy
    qz_ref[...] = qz


def _fps(px, py, pz, m):
    n = px.shape[1]
    out = jax.ShapeDtypeStruct((_B, m), jnp.float32)
    hb = _B // 2
    ispec = pl.BlockSpec((hb, n), lambda g: (g, 0))
    ospec = pl.BlockSpec((hb, m), lambda g: (g, 0))
    return pl.pallas_call(
        functools.partial(_fps_body, m=m, n=n),
        grid=(2,),
        in_specs=[ispec, ispec, ispec],
        out_specs=[ospec, ospec, ospec],
        out_shape=[out, out, out],
        compiler_params=_PARALLEL,
    )(px, py, pz)


def _seq_rowsum(x):
    s = x[0:1, :]
    for i in range(1, x.shape[0]):
        s = s + x[i : i + 1, :]
    return s


def _mlp_instnorm(h, wb_refs, seq=False):
    m = h.shape[0]
    for i in range(0, len(wb_refs), 2):
        w = wb_refs[i][...]
        bb = wb_refs[i + 1][...]
        h = jnp.dot(h, w, preferred_element_type=jnp.float32) + bb
        if seq:
            mu = _seq_rowsum(h) / m
            var = _seq_rowsum((h - mu) ** 2) / m
        else:
            mu = jnp.mean(h, axis=0, keepdims=True)
            var = jnp.mean((h - mu) ** 2, axis=0, keepdims=True)
        h = (h - mu) / jnp.sqrt(var + 1e-5)
        h = jnp.maximum(h, 0.0)
    return h


def _sa_body(fT_ref, q_ref, *refs, m, n, c_in, r2, kk, lane_layers):
    out_ref = refs[-1]
    wb = refs[:-1]
    cf = c_in + 3
    q = q_ref[0]
    qx = q[:, 0:1]
    qy = q[:, 1:2]
    qz = q[:, 2:3]
    pxr = fT_ref[0, c_in : c_in + 1, :]
    pyr = fT_ref[0, c_in + 1 : c_in + 2, :]
    pzr = fT_ref[0, c_in + 2 : c_in + 3, :]
    d2 = (qx - pxr) ** 2 + (qy - pyr) ** 2 + (qz - pzr) ** 2

    if kk < n:
        cnt_r2 = jnp.sum(jnp.where(d2 <= r2, 1, 0), axis=1, keepdims=True)

        def bisect(_):
            bits = jax.lax.bitcast_convert_type(d2, jnp.int32)

            def bis(_, lohi):
                lo, hi = lohi
                mid = lo + jax.lax.shift_right_logical(hi - lo, 1)
                cnt = jnp.sum(
                    jnp.where(bits <= mid, 1, 0), axis=1, keepdims=True
                )
                ge = cnt >= kk
                return jnp.where(ge, lo, mid + 1), jnp.where(ge, mid, hi)

            lo0 = jnp.zeros((m, 1), jnp.int32)
            hi0 = jnp.full((m, 1), _MAXF, jnp.int32)
            lo, _ = jax.lax.fori_loop(0, 31, bis, (lo0, hi0))
            return jax.lax.bitcast_convert_type(lo, jnp.float32)

        tk = jax.lax.cond(
            jnp.max(cnt_r2) > kk,
            bisect,
            lambda _: jnp.full((m, 1), jnp.float32(3.0e38)),
            None,
        )
        thr = jnp.minimum(tk, r2)
    else:
        thr = jnp.full((m, 1), r2, jnp.float32)

    lane_c = jax.lax.broadcasted_iota(jnp.int32, (m, cf), 1)

    def pc(c, acc):
        row = fT_ref[0, pl.ds(c, 1), :]
        red = jnp.max(
            jnp.where(d2 <= thr, row, _NEG), axis=1, keepdims=True
        )
        return jnp.where(lane_c == c, red, acc)

    h = jax.lax.fori_loop(0, cf, pc, jnp.zeros((m, cf), jnp.float32))
    qpad = (
        jnp.where(lane_c == c_in, qx, 0.0)
        + jnp.where(lane_c == c_in + 1, qy, 0.0)
        + jnp.where(lane_c == c_in + 2, qz, 0.0)
    )
    h = h - qpad
    if lane_layers:
        hT = jnp.transpose(h)
        for li in range(lane_layers):
            wT = wb[2 * li][...]
            bc = wb[2 * li + 1][...]
            hT = jnp.dot(wT, hT, preferred_element_type=jnp.float32) + bc
            mu = jnp.mean(hT, axis=1, keepdims=True)
            var = jnp.mean((hT - mu) ** 2, axis=1, keepdims=True)
            hT = (hT - mu) / jnp.sqrt(var + 1e-5)
            hT = jnp.maximum(hT, 0.0)
        h = jnp.transpose(hT)
    out_ref[0] = _mlp_instnorm(h, wb[2 * lane_layers :], seq=(m <= 12))


def _const_spec(shape):
    nd = len(shape)
    return pl.BlockSpec(shape, lambda b: (0,) * nd)


def _wb_specs_args(layers):
    specs, args = [], []
    for w, bb in layers:
        b2 = bb.reshape(1, -1)
        specs.append(_const_spec(w.shape))
        specs.append(_const_spec(b2.shape))
        args.append(w)
        args.append(b2)
    return specs, args


def _sa(fT, q, layers, r, kk, lane_layers=0):
    _, cf, n = fT.shape
    m = q.shape[1]
    c_in = cf - 3
    cout = layers[-1][0].shape[1]
    wspecs, wargs = [], []
    for li, (w, bb) in enumerate(layers):
        if li < lane_layers:
            wa, ba = w.T, bb.reshape(-1, 1)
        else:
            wa, ba = w, bb.reshape(1, -1)
        wspecs += [_const_spec(wa.shape), _const_spec(ba.shape)]
        wargs += [wa, ba]
    body = functools.partial(
        _sa_body, m=m, n=n, c_in=c_in, r2=r * r, kk=kk,
        lane_layers=lane_layers,
    )
    return pl.pallas_call(
        body,
        grid=(_B,),
        in_specs=[
            pl.BlockSpec((1, cf, n), lambda b: (b, 0, 0)),
            pl.BlockSpec((1, m, 3), lambda b: (b, 0, 0)),
            *wspecs,
        ],
        out_specs=pl.BlockSpec((1, m, cout), lambda b: (b, 0, 0)),
        out_shape=jax.ShapeDtypeStruct((_B, m, cout), jnp.float32),
        compiler_params=_PARALLEL,
    )(fT, q, *wargs)


def _fp_body(posT_ref, q_ref, x_ref, xs_ref, *refs, m, n, k, nwb):
    out_ref = refs[-1]
    wb = refs[:nwb]
    lin = refs[nwb:-1]
    q = q_ref[0]
    qx = q[:, 0:1]
    qy = q[:, 1:2]
    qz = q[:, 2:3]
    pxr = posT_ref[0, 0:1, :]
    pyr = posT_ref[0, 1:2, :]
    pzr = posT_ref[0, 2:3, :]
    d2 = (qx - pxr) ** 2 + (qy - pyr) ** 2 + (qz - pzr) ** 2

    lane = jax.lax.broadcasted_iota(jnp.int32, (m, n), 1)
    x = x_ref[0]
    d2cur = d2
    xi = None
    wsum = None
    for p in range(k):
        mv = jnp.min(d2cur, axis=1, keepdims=True)
        cand = jnp.where(d2cur == mv, lane, n)
        j = jnp.min(cand, axis=1, keepdims=True)
        oh = (lane == j).astype(jnp.float32)
        xp = jnp.dot(
            oh,
            x,
            preferred_element_type=jnp.float32,
            precision=jax.lax.Precision.HIGHEST,
        )
        wp = 1.0 / jnp.maximum(mv, 1e-16)
        xi = wp * xp if xi is None else xi + wp * xp
        wsum = wp if wsum is None else wsum + wp
        if p + 1 < k:
            d2cur = jnp.where(lane == j, jnp.inf, d2cur)
    xi = xi / wsum
    h = jnp.concatenate([xi, xs_ref[0]], axis=1)
    h = _mlp_instnorm(h, wb, seq=(m <= 12))
    if lin:
        w1, b1, w2, b2, w3, b3 = (r[...] for r in lin)
        h = jnp.maximum(jnp.dot(h, w1, preferred_element_type=jnp.float32) + b1, 0.0)
        h = jnp.dot(h, w2, preferred_element_type=jnp.float32) + b2
        h = jnp.dot(h, w3, preferred_element_type=jnp.float32) + b3
    out_ref[0] = h


def _fp(posT, q, x, xs, layers, k, lin=None):
    n = posT.shape[2]
    m = q.shape[1]
    c = x.shape[2]
    cs = xs.shape[2]
    cout = layers[-1][0].shape[1]
    wspecs, wargs = _wb_specs_args(layers)
    lspecs, largs = ([], [])
    if lin is not None:
        lspecs, largs = _wb_specs_args(lin)
        cout = lin[-1][0].shape[1]
    body = functools.partial(
        _fp_body, m=m, n=n, k=k, nwb=len(wspecs)
    )
    return pl.pallas_call(
        body,
        grid=(_B,),
        in_specs=[
            pl.BlockSpec((1, 3, n), lambda b: (b, 0, 0)),
            pl.BlockSpec((1, m, 3), lambda b: (b, 0, 0)),
            pl.BlockSpec((1, n, c), lambda b: (b, 0, 0)),
            pl.BlockSpec((1, m, cs), lambda b: (b, 0, 0)),
            *wspecs,
            *lspecs,
        ],
        out_specs=pl.BlockSpec((1, m, cout), lambda b: (b, 0, 0)),
        out_shape=jax.ShapeDtypeStruct((_B, m, cout), jnp.float32),
        compiler_params=_PARALLEL,
    )(posT, q, x, xs, *wargs, *largs)


def kernel(x, pos, batch, params):
    del batch
    xb = x.reshape(_B, _P, _IN)
    pb = pos.reshape(_B, _P, 3)
    px, py, pz = pb[:, :, 0], pb[:, :, 1], pb[:, :, 2]

    coords = [(px, py, pz)]
    for m in _MS:
        coords.append(_fps(*coords[-1], m))
    q3 = [jnp.stack(c, axis=-1) for c in coords]
    qT = [jnp.stack(c, axis=1) for c in coords]

    hs = []
    h = xb
    for i, (name, m, r) in enumerate(
        zip(["sa1", "sa2", "sa3", "sa4", "sa5"], _MS, _RS)
    ):
        n = h.shape[1]
        fT = jnp.concatenate([jnp.transpose(h, (0, 2, 1)), qT[i]], axis=1)
        h = _sa(
            fT, q3[i + 1], params[name], r, min(128, n),
            lane_layers=2 if i == 0 else 0,
        )
        hs.append(h)
    h1, h2, h3, h4, h5 = hs

    lin = [params["lin1"], params["lin2"], params["lin3"]]
    f5 = _fp(qT[5], q3[4], h5, h4, params["fp5"], 1)
    f4 = _fp(qT[4], q3[3], f5, h3, params["fp4"], 3)
    f3 = _fp(qT[3], q3[2], f4, h2, params["fp3"], 3)
    f2 = _fp(qT[2], q3[1], f3, h1, params["fp2"], 3)
    out = _fp(qT[1], q3[0], f2, xb, params["fp1"], 3, lin=lin)
    return out.reshape(_B * _P, _OUT)


# speedup vs baseline: 1.1067x; 1.1067x over previous
import functools

import jax
import jax.numpy as jnp
from jax.experimental import pallas as pl
from jax.experimental.pallas import tpu as pltpu

_PARALLEL = pltpu.CompilerParams(dimension_semantics=("parallel",))

_B, _P, _IN, _OUT = 16, 2048, 3, 13
_NEG = -1e30
_MS = [410, 123, 37, 12, 4]
_RS = [0.05, 0.1, 0.2, 0.4, 0.8]
_MAXF = 0x7F7FFFFF


def _fps_body(px_ref, py_ref, pz_ref, qx_ref, qy_ref, qz_ref, *, m, n):
    px = px_ref[...]
    py = py_ref[...]
    pz = pz_ref[...]
    b = px.shape[0]
    lane = jax.lax.broadcasted_iota(jnp.int32, (b, n), 1)
    lane_m = jax.lax.broadcasted_iota(jnp.int32, (b, m), 1)
    lx = px[:, 0:1]
    ly = py[:, 0:1]
    lz = pz[:, 0:1]
    qx0 = jnp.where(lane_m == 0, lx, 0.0)
    qy0 = jnp.where(lane_m == 0, ly, 0.0)
    qz0 = jnp.where(lane_m == 0, lz, 0.0)
    d0 = jnp.full((b, n), jnp.inf, dtype=jnp.float32)

    def body(i, carry):
        dists, cx, cy, cz, qx, qy, qz = carry
        d = (px - cx) ** 2 + (py - cy) ** 2 + (pz - cz) ** 2
        dists = jnp.minimum(dists, d)
        mx = jnp.max(dists, axis=1, keepdims=True)
        cand = jnp.where(dists == mx, lane, n)
        nxt = jnp.min(cand, axis=1, keepdims=True)
        oh = lane == nxt
        nx = jnp.sum(jnp.where(oh, px, 0.0), axis=1, keepdims=True)
        ny = jnp.sum(jnp.where(oh, py, 0.0), axis=1, keepdims=True)
        nz = jnp.sum(jnp.where(oh, pz, 0.0), axis=1, keepdims=True)
        qx = jnp.where(lane_m == i, nx, qx)
        qy = jnp.where(lane_m == i, ny, qy)
        qz = jnp.where(lane_m == i, nz, qz)
        return dists, nx, ny, nz, qx, qy, qz

    _, _, _, _, qx, qy, qz = jax.lax.fori_loop(
        1, m, body, (d0, lx, ly, lz, qx0, qy0, qz0), unroll=2
    )
    qx_ref[...] = qx
    qy_ref[...] = qy
    qz_ref[...] = qz


def _fps(px, py, pz, m):
    n = px.shape[1]
    out = jax.ShapeDtypeStruct((_B, m), jnp.float32)
    return pl.pallas_call(
        functools.partial(_fps_body, m=m, n=n),
        out_shape=[out, out, out],
    )(px, py, pz)


def _seq_rowsum(x):
    s = x[0:1, :]
    for i in range(1, x.shape[0]):
        s = s + x[i : i + 1, :]
    return s


def _mlp_instnorm(h, wb_refs, seq=False):
    m = h.shape[0]
    for i in range(0, len(wb_refs), 2):
        w = wb_refs[i][...]
        bb = wb_refs[i + 1][...]
        h = jnp.dot(h, w, preferred_element_type=jnp.float32) + bb
        if seq:
            mu = _seq_rowsum(h) / m
            var = _seq_rowsum((h - mu) ** 2) / m
        else:
            mu = jnp.mean(h, axis=0, keepdims=True)
            var = jnp.mean((h - mu) ** 2, axis=0, keepdims=True)
        h = (h - mu) / jnp.sqrt(var + 1e-5)
        h = jnp.maximum(h, 0.0)
    return h


def _sa_body(fT_ref, q_ref, *refs, m, n, c_in, r2, kk, lane_layers):
    out_ref = refs[-1]
    wb = refs[:-1]
    cf = c_in + 3
    q = q_ref[0]
    qx = q[:, 0:1]
    qy = q[:, 1:2]
    qz = q[:, 2:3]
    pxr = fT_ref[0, c_in : c_in + 1, :]
    pyr = fT_ref[0, c_in + 1 : c_in + 2, :]
    pzr = fT_ref[0, c_in + 2 : c_in + 3, :]
    d2 = (qx - pxr) ** 2 + (qy - pyr) ** 2 + (qz - pzr) ** 2

    if kk < n:
        cnt_r2 = jnp.sum(jnp.where(d2 <= r2, 1, 0), axis=1, keepdims=True)

        def bisect(_):
            bits = jax.lax.bitcast_convert_type(d2, jnp.int32)

            def bis(_, lohi):
                lo, hi = lohi
                mid = lo + jax.lax.shift_right_logical(hi - lo, 1)
                cnt = jnp.sum(
                    jnp.where(bits <= mid, 1, 0), axis=1, keepdims=True
                )
                ge = cnt >= kk
                return jnp.where(ge, lo, mid + 1), jnp.where(ge, mid, hi)

            lo0 = jnp.zeros((m, 1), jnp.int32)
            hi0 = jnp.full((m, 1), _MAXF, jnp.int32)
            lo, _ = jax.lax.fori_loop(0, 31, bis, (lo0, hi0))
            return jax.lax.bitcast_convert_type(lo, jnp.float32)

        tk = jax.lax.cond(
            jnp.max(cnt_r2) > kk,
            bisect,
            lambda _: jnp.full((m, 1), jnp.float32(3.0e38)),
            None,
        )
        thr = jnp.minimum(tk, r2)
    else:
        thr = jnp.full((m, 1), r2, jnp.float32)

    lane_c = jax.lax.broadcasted_iota(jnp.int32, (m, cf), 1)

    def pc(c, acc):
        row = fT_ref[0, pl.ds(c, 1), :]
        red = jnp.max(
            jnp.where(d2 <= thr, row, _NEG), axis=1, keepdims=True
        )
        return jnp.where(lane_c == c, red, acc)

    h = jax.lax.fori_loop(0, cf, pc, jnp.zeros((m, cf), jnp.float32))
    qpad = (
        jnp.where(lane_c == c_in, qx, 0.0)
        + jnp.where(lane_c == c_in + 1, qy, 0.0)
        + jnp.where(lane_c == c_in + 2, qz, 0.0)
    )
    h = h - qpad
    if lane_layers:
        hT = jnp.transpose(h)
        for li in range(lane_layers):
            wT = wb[2 * li][...]
            bc = wb[2 * li + 1][...]
            hT = jnp.dot(wT, hT, preferred_element_type=jnp.float32) + bc
            mu = jnp.mean(hT, axis=1, keepdims=True)
            var = jnp.mean((hT - mu) ** 2, axis=1, keepdims=True)
            hT = (hT - mu) / jnp.sqrt(var + 1e-5)
            hT = jnp.maximum(hT, 0.0)
        h = jnp.transpose(hT)
    out_ref[0] = _mlp_instnorm(h, wb[2 * lane_layers :], seq=(m <= 12))


def _const_spec(shape):
    nd = len(shape)
    return pl.BlockSpec(shape, lambda b: (0,) * nd)


def _wb_specs_args(layers):
    specs, args = [], []
    for w, bb in layers:
        b2 = bb.reshape(1, -1)
        specs.append(_const_spec(w.shape))
        specs.append(_const_spec(b2.shape))
        args.append(w)
        args.append(b2)
    return specs, args


def _sa(fT, q, layers, r, kk, lane_layers=0):
    _, cf, n = fT.shape
    m = q.shape[1]
    c_in = cf - 3
    cout = layers[-1][0].shape[1]
    wspecs, wargs = [], []
    for li, (w, bb) in enumerate(layers):
        if li < lane_layers:
            wa, ba = w.T, bb.reshape(-1, 1)
        else:
            wa, ba = w, bb.reshape(1, -1)
        wspecs += [_const_spec(wa.shape), _const_spec(ba.shape)]
        wargs += [wa, ba]
    body = functools.partial(
        _sa_body, m=m, n=n, c_in=c_in, r2=r * r, kk=kk,
        lane_layers=lane_layers,
    )
    return pl.pallas_call(
        body,
        grid=(_B,),
        in_specs=[
            pl.BlockSpec((1, cf, n), lambda b: (b, 0, 0)),
            pl.BlockSpec((1, m, 3), lambda b: (b, 0, 0)),
            *wspecs,
        ],
        out_specs=pl.BlockSpec((1, m, cout), lambda b: (b, 0, 0)),
        out_shape=jax.ShapeDtypeStruct((_B, m, cout), jnp.float32),
        compiler_params=_PARALLEL,
    )(fT, q, *wargs)


def _fp_body(posT_ref, q_ref, x_ref, xs_ref, *refs, m, n, k, nwb):
    out_ref = refs[-1]
    wb = refs[:nwb]
    lin = refs[nwb:-1]
    q = q_ref[0]
    qx = q[:, 0:1]
    qy = q[:, 1:2]
    qz = q[:, 2:3]
    pxr = posT_ref[0, 0:1, :]
    pyr = posT_ref[0, 1:2, :]
    pzr = posT_ref[0, 2:3, :]
    d2 = (qx - pxr) ** 2 + (qy - pyr) ** 2 + (qz - pzr) ** 2

    lane = jax.lax.broadcasted_iota(jnp.int32, (m, n), 1)
    x = x_ref[0]
    d2cur = d2
    xi = None
    wsum = None
    for p in range(k):
        mv = jnp.min(d2cur, axis=1, keepdims=True)
        cand = jnp.where(d2cur == mv, lane, n)
        j = jnp.min(cand, axis=1, keepdims=True)
        oh = (lane == j).astype(jnp.float32)
        xp = jnp.dot(
            oh,
            x,
            preferred_element_type=jnp.float32,
            precision=jax.lax.Precision.HIGHEST,
        )
        wp = 1.0 / jnp.maximum(mv, 1e-16)
        xi = wp * xp if xi is None else xi + wp * xp
        wsum = wp if wsum is None else wsum + wp
        if p + 1 < k:
            d2cur = jnp.where(lane == j, jnp.inf, d2cur)
    xi = xi / wsum
    h = jnp.concatenate([xi, xs_ref[0]], axis=1)
    h = _mlp_instnorm(h, wb, seq=(m <= 12))
    if lin:
        w1, b1, w2, b2, w3, b3 = (r[...] for r in lin)
        h = jnp.maximum(jnp.dot(h, w1, preferred_element_type=jnp.float32) + b1, 0.0)
        h = jnp.dot(h, w2, preferred_element_type=jnp.float32) + b2
        h = jnp.dot(h, w3, preferred_element_type=jnp.float32) + b3
    out_ref[0] = h


def _fp(posT, q, x, xs, layers, k, lin=None):
    n = posT.shape[2]
    m = q.shape[1]
    c = x.shape[2]
    cs = xs.shape[2]
    cout = layers[-1][0].shape[1]
    wspecs, wargs = _wb_specs_args(layers)
    lspecs, largs = ([], [])
    if lin is not None:
        lspecs, largs = _wb_specs_args(lin)
        cout = lin[-1][0].shape[1]
    body = functools.partial(
        _fp_body, m=m, n=n, k=k, nwb=len(wspecs)
    )
    return pl.pallas_call(
        body,
        grid=(_B,),
        in_specs=[
            pl.BlockSpec((1, 3, n), lambda b: (b, 0, 0)),
            pl.BlockSpec((1, m, 3), lambda b: (b, 0, 0)),
            pl.BlockSpec((1, n, c), lambda b: (b, 0, 0)),
            pl.BlockSpec((1, m, cs), lambda b: (b, 0, 0)),
            *wspecs,
            *lspecs,
        ],
        out_specs=pl.BlockSpec((1, m, cout), lambda b: (b, 0, 0)),
        out_shape=jax.ShapeDtypeStruct((_B, m, cout), jnp.float32),
        compiler_params=_PARALLEL,
    )(posT, q, x, xs, *wargs, *largs)


def kernel(x, pos, batch, params):
    del batch
    xb = x.reshape(_B, _P, _IN)
    pb = pos.reshape(_B, _P, 3)
    px, py, pz = pb[:, :, 0], pb[:, :, 1], pb[:, :, 2]

    coords = [(px, py, pz)]
    for m in _MS:
        coords.append(_fps(*coords[-1], m))
    q3 = [jnp.stack(c, axis=-1) for c in coords]
    qT = [jnp.stack(c, axis=1) for c in coords]

    hs = []
    h = xb
    for i, (name, m, r) in enumerate(
        zip(["sa1", "sa2", "sa3", "sa4", "sa5"], _MS, _RS)
    ):
        n = h.shape[1]
        fT = jnp.concatenate([jnp.transpose(h, (0, 2, 1)), qT[i]], axis=1)
        h = _sa(
            fT, q3[i + 1], params[name], r, min(128, n),
            lane_layers=2 if i == 0 else 0,
        )
        hs.append(h)
    h1, h2, h3, h4, h5 = hs

    lin = [params["lin1"], params["lin2"], params["lin3"]]
    f5 = _fp(qT[5], q3[4], h5, h4, params["fp5"], 1)
    f4 = _fp(qT[4], q3[3], f5, h3, params["fp4"], 3)
    f3 = _fp(qT[3], q3[2], f4, h2, params["fp3"], 3)
    f2 = _fp(qT[2], q3[1], f3, h1, params["fp2"], 3)
    out = _fp(qT[1], q3[0], f2, xb, params["fp1"], 3, lin=lin)
    return out.reshape(_B * _P, _OUT)
